# baseline (speedup 1.0000x reference)
import functools

import jax
import jax.numpy as jnp
from jax.experimental import pallas as pl
from jax.experimental.pallas import tpu as pltpu
from jax.experimental.pallas import tpu_sc as plsc

SEQ = 2048
ENC = 512
HID = 1024
E = 8
B = 8192

TBG = 256
T = B // TBG
WMAX = T + E - 1
CHUNK = 8
CW = SEQ // CHUNK
NR = B * CHUNK
SC_WIN = 128
RW = SC_WIN // CHUNK


@functools.cache
def _vector_mesh():
    return plsc.VectorSubcoreMesh(
        core_axis_name="core", subcore_axis_name="subcore")


TBC = 256
NC = B // TBC


def _routing_kernel(ids_ref, dest_ref, off_ref, oh_ref, cs_ref, tot_ref):
    ids = ids_ref[...]
    lane = jax.lax.broadcasted_iota(jnp.int32, (B, E), 1)
    oh = (ids == lane).astype(jnp.float32)
    oh_ref[...] = oh
    r = jax.lax.broadcasted_iota(jnp.int32, (TBC, TBC), 0)
    c = jax.lax.broadcasted_iota(jnp.int32, (TBC, TBC), 1)
    ltri = (c <= r).astype(jnp.float32)

    for ci in range(NC):
        chunk = oh_ref[pl.ds(ci * TBC, TBC), :]
        cs = jnp.dot(ltri, chunk, preferred_element_type=jnp.float32)
        cs_ref[pl.ds(ci * TBC, TBC), :] = cs
        tot_ref[ci, :] = cs[TBC - 1, :]

    rc = jax.lax.broadcasted_iota(jnp.int32, (NC, NC), 0)
    cc = jax.lax.broadcasted_iota(jnp.int32, (NC, NC), 1)
    strict = (cc < rc).astype(jnp.float32)
    tot = tot_ref[...]
    carry = jnp.dot(strict, tot, preferred_element_type=jnp.float32)
    counts = jnp.sum(tot, axis=0, keepdims=True)

    rowc = jax.lax.broadcasted_iota(jnp.int32, (B, NC), 0) // TBC
    sel = (rowc == jax.lax.broadcasted_iota(jnp.int32, (B, NC), 1)
           ).astype(jnp.float32)
    carry_rows = jnp.dot(sel, carry, preferred_element_type=jnp.float32,
                         precision=jax.lax.Precision.HIGHEST)
    csum = cs_ref[...] + carry_rows
    rank = jnp.sum(oh * csum, axis=1, keepdims=True) - 1.0
    re = jax.lax.broadcasted_iota(jnp.int32, (E, E), 0)
    ce = jax.lax.broadcasted_iota(jnp.int32, (E, E), 1)
    m = (re < ce).astype(jnp.float32)
    offs = jnp.dot(counts, m, preferred_element_type=jnp.float32,
                   precision=jax.lax.Precision.HIGHEST)
    dest = rank + jnp.sum(oh * offs, axis=1, keepdims=True)
    dest_ref[...] = dest.astype(jnp.int32)
    off_ref[...] = offs.astype(jnp.int32)


def _routing(ids):
    return pl.pallas_call(
        _routing_kernel,
        out_shape=(jax.ShapeDtypeStruct((B, 1), jnp.int32),
                   jax.ShapeDtypeStruct((1, E), jnp.int32)),
        scratch_shapes=[pltpu.VMEM((B, E), jnp.float32),
                        pltpu.VMEM((B, E), jnp.float32),
                        pltpu.VMEM((NC, E), jnp.float32)],
    )(ids)


def _dispatch(x, idx):
    @functools.partial(
        pl.kernel,
        out_type=tuple(jax.ShapeDtypeStruct((B, CW), jnp.float32)
                       for _ in range(CHUNK)),
        mesh=_vector_mesh())
    def run(x_hbm, i_hbm, *o_hbms):
        def body(x_vmem, i_vmem):
            for k in range(CHUNK):
                pltpu.sync_copy(
                    x_vmem.at[:, pl.ds(k * CW, CW)],
                    o_hbms[k].at[i_vmem.at[0, pl.ds(k * RW, RW)]])

        pltpu.emit_pipeline(
            body,
            grid=(B // RW,),
            in_specs=[pl.BlockSpec((RW, SEQ + 1), lambda i: (i, 0)),
                      pl.BlockSpec((1, SC_WIN), lambda i: (0, i))],
            out_specs=[],
            core_axis_name=("core", "subcore"),
            dimension_semantics=(pltpu.PARALLEL,),
        )(x_hbm, i_hbm)

    return run(x, idx)


def _combine(ys_list, idx):
    @functools.partial(
        pl.kernel,
        out_type=jax.ShapeDtypeStruct((B, SEQ), jnp.float32),
        mesh=_vector_mesh())
    def run(i_hbm, *ys_and_out):
        ys_hbms, o_hbm = ys_and_out[:CHUNK], ys_and_out[CHUNK]

        def body(i_vmem, o_vmem):
            for k in range(CHUNK):
                pltpu.sync_copy(
                    ys_hbms[k].at[i_vmem.at[0, pl.ds(k * RW, RW)]],
                    o_vmem.at[:, pl.ds(k * CW, CW)])

        pltpu.emit_pipeline(
            body,
            grid=(B // RW,),
            in_specs=[pl.BlockSpec((1, SC_WIN), lambda i: (0, i))],
            out_specs=[pl.BlockSpec((RW, SEQ), lambda i: (i, 0))],
            core_axis_name=("core", "subcore"),
            dimension_semantics=(pltpu.PARALLEL,),
        )(i_hbm, o_hbm)

    return run(idx, *ys_list)


def _metadata(offsets):
    o = offsets
    ends = jnp.concatenate([o[1:], jnp.full((1,), B, jnp.int32)])
    counts = ends - o
    f = o // TBG
    l = (ends - 1) // TBG
    tpg = jnp.where(counts > 0, l - f + 1, 0)
    cw = jnp.concatenate([jnp.zeros((1,), jnp.int32),
                          jnp.cumsum(tpg).astype(jnp.int32)])
    total = cw[E]
    w = jnp.arange(WMAX, dtype=jnp.int32)
    gid = jnp.sum((w[:, None] >= cw[None, 1:]).astype(jnp.int32), axis=1)
    gid = jnp.minimum(gid, E - 1)
    tile = f[gid] + (w - cw[gid])
    valid = w < total
    tile = jnp.where(valid, tile, T - 1)
    start = jnp.where(valid, jnp.maximum(o[gid], tile * TBG), 0)
    end = jnp.where(valid, jnp.minimum(ends[gid], (tile + 1) * TBG), 0)
    first = (start == tile * TBG).astype(jnp.int32)
    return tile, gid, start, end, first


def _mlp_kernel(tl_ref, gd_ref, st_ref, en_ref, fr_ref,
                *refs):
    xs_refs = refs[:CHUNK]
    w1_ref, b1_ref, w2_ref, b2_ref, w3_ref, b3_ref, w4_ref, b4_ref = \
        refs[CHUNK:CHUNK + 8]
    out_refs = refs[CHUNK + 8:]
    w = pl.program_id(0)
    start, end, first = st_ref[w], en_ref[w], fr_ref[w]

    @pl.when(start < end)
    def _():
        xv = jnp.concatenate([xs_refs[k][...] for k in range(CHUNK)], axis=1)
        h = jnp.maximum(
            jnp.dot(xv, w1_ref[0], preferred_element_type=jnp.float32)
            + b1_ref[0], 0.0)
        z = jnp.maximum(
            jnp.dot(h, w2_ref[0], preferred_element_type=jnp.float32)
            + b2_ref[0], 0.0)
        h2 = jnp.maximum(
            jnp.dot(z, w3_ref[0], preferred_element_type=jnp.float32)
            + b3_ref[0], 0.0)
        y = (jnp.dot(h2, w4_ref[0], preferred_element_type=jnp.float32)
             + b4_ref[0])
        rows = (tl_ref[w] * TBG
                + jax.lax.broadcasted_iota(jnp.int32, (TBG, 1), 0))
        m = (rows >= start) & (rows < end)
        for k in range(CHUNK):
            yk = y[:, k * CW:(k + 1) * CW]

            @pl.when(first == 1)
            def _(yk=yk, k=k):
                out_refs[k][...] = jnp.where(m, yk, 0.0)

            @pl.when(first == 0)
            def _(yk=yk, k=k):
                out_refs[k][...] = jnp.where(m, yk, out_refs[k][...])


def _grouped_mlp(xs_list, meta, W1, b1, W2, b2, W3, b3, W4, b4):
    tile, gid, start, end, first = meta
    grid_spec = pltpu.PrefetchScalarGridSpec(
        num_scalar_prefetch=5,
        grid=(WMAX,),
        in_specs=[
            *[pl.BlockSpec((TBG, CW),
                           lambda w, tl, gd, st, en, fr: (tl[w], 0))
              for _ in range(CHUNK)],
            pl.BlockSpec((1, SEQ, HID),
                         lambda w, tl, gd, st, en, fr: (gd[w], 0, 0)),
            pl.BlockSpec((1, 1, HID),
                         lambda w, tl, gd, st, en, fr: (gd[w], 0, 0)),
            pl.BlockSpec((1, HID, ENC),
                         lambda w, tl, gd, st, en, fr: (gd[w], 0, 0)),
            pl.BlockSpec((1, 1, ENC),
                         lambda w, tl, gd, st, en, fr: (gd[w], 0, 0)),
            pl.BlockSpec((1, ENC, HID),
                         lambda w, tl, gd, st, en, fr: (gd[w], 0, 0)),
            pl.BlockSpec((1, 1, HID),
                         lambda w, tl, gd, st, en, fr: (gd[w], 0, 0)),
            pl.BlockSpec((1, HID, SEQ),
                         lambda w, tl, gd, st, en, fr: (gd[w], 0, 0)),
            pl.BlockSpec((1, 1, SEQ),
                         lambda w, tl, gd, st, en, fr: (gd[w], 0, 0)),
        ],
        out_specs=[pl.BlockSpec((TBG, CW),
                               lambda w, tl, gd, st, en, fr: (tl[w], 0))
                   for _ in range(CHUNK)],
    )
    return pl.pallas_call(
        _mlp_kernel,
        grid_spec=grid_spec,
        out_shape=tuple(jax.ShapeDtypeStruct((B, CW), jnp.float32)
                        for _ in range(CHUNK)),
    )(tile, gid, start, end, first, *xs_list,
      W1, b1.reshape(E, 1, HID), W2, b2.reshape(E, 1, ENC),
      W3, b3.reshape(E, 1, HID), W4, b4.reshape(E, 1, SEQ))


@jax.jit
def kernel(x, W1, b1, W2, b2, W3, b3, W4, b4):
    ids = x[:, SEQ].astype(jnp.int32).reshape(B, 1)
    dest, off = _routing(ids)
    dest_r = dest.reshape(B // RW, RW)
    idx = jnp.broadcast_to(dest_r[:, None, :],
                           (B // RW, CHUNK, RW)).reshape(1, NR)
    meta = _metadata(off[0])
    xs_list = _dispatch(x, idx)
    ys_list = _grouped_mlp(xs_list, meta, W1, b1, W2, b2, W3, b3, W4, b4)
    return _combine(ys_list, idx)
